# Initial kernel scaffold; baseline (speedup 1.0000x reference)
import jax
import jax.numpy as jnp
from jax.experimental import pallas as pl


def kernel(query, support, W1, g1, b1, rm1, rv1, W2, g2, b2, rm2, rv2, W3, g3, b3, rm3, rv3, W4, g4, b4, rm4, rv4):
    raise NotImplementedError("write your pallas kernel here")


# speedup vs baseline: 115.7507x; 115.7507x over previous
import jax
import jax.numpy as jnp
from jax import lax
from jax.experimental import pallas as pl


def _enc_layer(x, W, g, b, rm, rv, pool):
    x = lax.conv_general_dilated(x, W, (1, 1), 'SAME',
                                 dimension_numbers=('NCHW', 'OIHW', 'NCHW'))
    x = (x - rm[None, :, None, None]) * lax.rsqrt(rv[None, :, None, None] + 1e-5) \
        * g[None, :, None, None] + b[None, :, None, None]
    x = jax.nn.leaky_relu(x, 0.2)
    if pool:
        x = lax.reduce_window(x, -jnp.inf, lax.max, (1, 1, 2, 2), (1, 1, 2, 2), 'VALID')
    return x


def _vote_kernel(q_ref, s_ref, out_ref):
    q = q_ref[0]
    qn = q / jnp.maximum(jnp.sqrt(jnp.sum(q * q, axis=1, keepdims=True)), 1e-12)
    way = s_ref.shape[1]
    accs = []
    for c in range(way):
        s = s_ref[0, c]
        sn = s / jnp.maximum(jnp.sqrt(jnp.sum(s * s, axis=1, keepdims=True)), 1e-12)
        sim = lax.dot_general(qn, sn, (((1,), (1,)), ((), ())),
                              preferred_element_type=jnp.float32)
        m_dim = sim.shape[1]
        col = lax.broadcasted_iota(jnp.int32, sim.shape, 1)
        acc = jnp.zeros((1, 1), jnp.float32)
        for _ in range(3):
            m = jnp.max(sim, axis=1, keepdims=True)
            acc = acc + jnp.sum(m, axis=0, keepdims=True)
            am = jnp.min(jnp.where(sim == m, col, m_dim), axis=1, keepdims=True)
            sim = jnp.where(col == am, -jnp.inf, sim)
        accs.append(acc)
    out_ref[0] = jnp.concatenate(accs, axis=1)


def kernel(query, support, W1, g1, b1, rm1, rv1, W2, g2, b2, rm2, rv2,
           W3, g3, b3, rm3, rv3, W4, g4, b4, rm4, rv4):
    B, NQ, C, H, W = query.shape
    _, Way, Shot, _, _, _ = support.shape

    def encoder(x):
        x = _enc_layer(x, W1, g1, b1, rm1, rv1, True)
        x = _enc_layer(x, W2, g2, b2, rm2, rv2, True)
        x = _enc_layer(x, W3, g3, b3, rm3, rv3, False)
        x = _enc_layer(x, W4, g4, b4, rm4, rv4, False)
        return x

    qf = encoder(query.reshape(-1, C, H, W))
    sf = encoder(support.reshape(-1, C, H, W))
    D, h, w = qf.shape[1], qf.shape[2], qf.shape[3]
    L = h * w
    M = Shot * h * w
    q_local = qf.reshape(B * NQ, D, L).transpose(0, 2, 1)
    s_local = sf.reshape(B, Way, Shot, D, L).transpose(0, 1, 2, 4, 3) \
                .reshape(B, Way, M, D)

    scores = pl.pallas_call(
        _vote_kernel,
        grid=(B * NQ,),
        in_specs=[
            pl.BlockSpec((1, L, D), lambda i: (i, 0, 0)),
            pl.BlockSpec((1, Way, M, D), lambda i: (i // NQ, 0, 0, 0)),
        ],
        out_specs=pl.BlockSpec((1, 1, Way), lambda i: (i, 0, 0)),
        out_shape=jax.ShapeDtypeStruct((B * NQ, 1, Way), jnp.float32),
    )(q_local, s_local)
    return scores.reshape(B * NQ, Way)
